# baseline (speedup 1.0000x reference)
import numpy as np
import jax
import jax.numpy as jnp
from jax.experimental import pallas as pl
from jax.experimental.pallas import tpu as pltpu

BR = 400
BG = 800
CW = 2048
NPAD = 10240
APAD = 10400
_P = jax.lax.Precision.DEFAULT


def _build_schedule(n: int) -> np.ndarray:
    nbg = -(-n // BG)
    nbc = NPAD // CW
    rows = []
    for g in range(nbg):
        c0 = (BG * g) // CW
        for c in range(c0, nbc):
            rows.append((g, c, max(BG * g - CW * c, 0),
                         1 if c == nbc - 1 else 0))
    return np.asarray(rows, dtype=np.int32).T.copy()


def _support_body(x_ref, w1_ref, s_ref):
    s_ref[...] = jnp.dot(x_ref[...], w1_ref[...], precision=_P,
                         preferred_element_type=jnp.float32)


def _phase_a_body(adj_ref, s_ref, b1_ref, h_ref, acc_ref, hprev_ref):
    i = pl.program_id(0)
    nsteps = pl.num_programs(0)

    @pl.when(i == 0)
    def _():
        h_ref[...] = jnp.zeros_like(h_ref)
        acc_ref[...] = jnp.zeros_like(acc_ref)

    a = adj_ref[...]
    acc_ref[pl.ds(i * BR, BR), :] = jnp.dot(
        a, h_ref[: a.shape[1], :], precision=_P,
        preferred_element_type=jnp.float32)
    h_i = jnp.maximum(
        jnp.dot(a, s_ref[...], precision=_P,
                preferred_element_type=jnp.float32) + b1_ref[...], 0.0)

    @pl.when(i % 2 == 0)
    def _():
        hprev_ref[...] = h_i

    @pl.when(i % 2 == 1)
    def _():
        h_ref[pl.ds((i - 1) * BR, BR), :] = hprev_ref[...]
        h_ref[pl.ds(i * BR, BR), :] = h_i

    @pl.when((i == nsteps - 1) & (i % 2 == 0))
    def _():
        h_ref[pl.ds(i * BR, BR), :] = h_i


def _phase_b_body(n, sref, adj_ref, h_ref, acc_in_ref, w2_ref, b2_ref,
                  out_ref, acc_ref):
    t = pl.program_id(0)

    @pl.when(t == 0)
    def _():
        acc_ref[...] = jnp.zeros_like(acc_ref)

    gg = sref[0, t]
    cc = sref[1, t]
    lo_rel = sref[2, t]
    rid = jax.lax.broadcasted_iota(jnp.int32, (CW, 1), 0)
    hs = jnp.where(rid < lo_rel, 0.0, h_ref[pl.ds(cc * CW, CW), :])

    @pl.when(sref[3, t] == 0)
    def _():
        acc_ref[...] += jnp.dot(adj_ref[...], hs, precision=_P,
                                preferred_element_type=jnp.float32)

    @pl.when(sref[3, t] == 1)
    def _():
        c_last = NPAD // CW - 1
        k0 = (n // 128) * 128 - c_last * CW
        rem = n - c_last * CW - k0
        col = jax.lax.broadcasted_iota(jnp.int32, (BG, 128), 1)
        a_rem = jnp.where(col < rem, adj_ref[:, k0:k0 + 128], 0.0)
        acc_ref[...] += (
            jnp.dot(adj_ref[:, :k0], hs[:k0, :], precision=_P,
                    preferred_element_type=jnp.float32)
            + jnp.dot(a_rem, hs[k0:k0 + 128, :], precision=_P,
                      preferred_element_type=jnp.float32))

        u = jnp.dot(acc_ref[...] + acc_in_ref[pl.ds(gg * BG, BG), :],
                    w2_ref[...], precision=_P,
                    preferred_element_type=jnp.float32) + b2_ref[...]
        m = jnp.max(u, axis=1, keepdims=True)
        lse = jnp.log(jnp.sum(jnp.exp(u - m), axis=1, keepdims=True)) + m
        out_ref[...] = u - lse
        acc_ref[...] = jnp.zeros_like(acc_ref)


def kernel(x, adj, W1, b1, W2, b2):
    n, nfeat = x.shape
    nhid = W1.shape[1]
    nclass = W2.shape[1]
    b1r = b1.reshape(1, nhid)
    b2r = b2.reshape(1, nclass)

    support = pl.pallas_call(
        _support_body,
        out_shape=jax.ShapeDtypeStruct((n, nhid), jnp.float32),
    )(x, W1)

    h_pad, acc = pl.pallas_call(
        _phase_a_body,
        grid=(n // BR,),
        in_specs=[
            pl.BlockSpec((BR, n), lambda i: (i, 0)),
            pl.BlockSpec((n, nhid), lambda i: (0, 0)),
            pl.BlockSpec((1, nhid), lambda i: (0, 0)),
        ],
        out_specs=[
            pl.BlockSpec((NPAD, nhid), lambda i: (0, 0)),
            pl.BlockSpec((APAD, nhid), lambda i: (0, 0)),
        ],
        out_shape=[
            jax.ShapeDtypeStruct((NPAD, nhid), jnp.float32),
            jax.ShapeDtypeStruct((APAD, nhid), jnp.float32),
        ],
        scratch_shapes=[pltpu.VMEM((BR, nhid), jnp.float32)],
    )(adj, support, b1r)

    sched = jnp.asarray(_build_schedule(n))
    tsteps = sched.shape[1]

    grid_spec = pltpu.PrefetchScalarGridSpec(
        num_scalar_prefetch=1,
        grid=(tsteps,),
        in_specs=[
            pl.BlockSpec((BG, CW), lambda t, s: (s[0, t], s[1, t])),
            pl.BlockSpec((NPAD, nhid), lambda t, s: (0, 0)),
            pl.BlockSpec((APAD, nhid), lambda t, s: (0, 0)),
            pl.BlockSpec((nhid, nclass), lambda t, s: (0, 0)),
            pl.BlockSpec((1, nclass), lambda t, s: (0, 0)),
        ],
        out_specs=pl.BlockSpec((BG, nclass), lambda t, s: (s[0, t], 0)),
        scratch_shapes=[pltpu.VMEM((BG, nhid), jnp.float32)],
    )

    import functools
    out = pl.pallas_call(
        functools.partial(_phase_b_body, n),
        grid_spec=grid_spec,
        out_shape=jax.ShapeDtypeStruct((n, nclass), jnp.float32),
    )(sched, adj, h_pad, acc, W2, b2r)

    return out


# speedup vs baseline: 1.0220x; 1.0220x over previous
import functools

import numpy as np
import jax
import jax.numpy as jnp
from jax.experimental import pallas as pl
from jax.experimental.pallas import tpu as pltpu

BR = 400
BG = 800
CW = 2048
NPAD = 10240
APAD = 10400
_P = jax.lax.Precision.DEFAULT


def _build_schedule(n: int) -> np.ndarray:
    nbg = -(-n // BG)
    nbc = NPAD // CW
    rows = []
    for g in range(nbg):
        c0 = (BG * g) // CW
        for c in range(c0, nbc):
            rows.append((g, c, max(BG * g - CW * c, 0),
                         1 if c == nbc - 1 else 0))
    return np.asarray(rows, dtype=np.int32).T.copy()


def _phase_a_body(adj_ref, x_ref, w1_ref, b1_ref, h_ref, acc_ref,
                  s_ref, hprev_ref):
    i = pl.program_id(0)
    nsteps = pl.num_programs(0)

    @pl.when(i == 0)
    def _():
        h_ref[...] = jnp.zeros_like(h_ref)
        acc_ref[...] = jnp.zeros_like(acc_ref)
        s_ref[...] = jnp.dot(x_ref[...], w1_ref[...], precision=_P,
                             preferred_element_type=jnp.float32)

    a = adj_ref[...]
    acc_ref[pl.ds(i * BR, BR), :] = jnp.dot(
        a, h_ref[: a.shape[1], :], precision=_P,
        preferred_element_type=jnp.float32)
    h_i = jnp.maximum(
        jnp.dot(a, s_ref[...], precision=_P,
                preferred_element_type=jnp.float32) + b1_ref[...], 0.0)

    @pl.when(i % 2 == 0)
    def _():
        hprev_ref[...] = h_i

    @pl.when(i % 2 == 1)
    def _():
        h_ref[pl.ds((i - 1) * BR, BR), :] = hprev_ref[...]
        h_ref[pl.ds(i * BR, BR), :] = h_i

    @pl.when((i == nsteps - 1) & (i % 2 == 0))
    def _():
        h_ref[pl.ds(i * BR, BR), :] = h_i


def _phase_b_body(n, sref, adj_ref, h_ref, acc_in_ref, w2_ref, b2_ref,
                  out_ref, acc_ref):
    t = pl.program_id(0)

    @pl.when(t == 0)
    def _():
        acc_ref[...] = jnp.zeros_like(acc_ref)

    gg = sref[0, t]
    cc = sref[1, t]
    lo_rel = sref[2, t]
    rid = jax.lax.broadcasted_iota(jnp.int32, (CW, 1), 0)
    hs = jnp.where(rid < lo_rel, 0.0, h_ref[pl.ds(cc * CW, CW), :])

    @pl.when(sref[3, t] == 0)
    def _():
        acc_ref[...] += jnp.dot(adj_ref[...], hs, precision=_P,
                                preferred_element_type=jnp.float32)

    @pl.when(sref[3, t] == 1)
    def _():
        c_last = NPAD // CW - 1
        k0 = (n // 128) * 128 - c_last * CW
        rem = n - c_last * CW - k0
        col = jax.lax.broadcasted_iota(jnp.int32, (BG, 128), 1)
        a_rem = jnp.where(col < rem, adj_ref[:, k0:k0 + 128], 0.0)
        acc_ref[...] += (
            jnp.dot(adj_ref[:, :k0], hs[:k0, :], precision=_P,
                    preferred_element_type=jnp.float32)
            + jnp.dot(a_rem, hs[k0:k0 + 128, :], precision=_P,
                      preferred_element_type=jnp.float32))

        u = jnp.dot(acc_ref[...] + acc_in_ref[pl.ds(gg * BG, BG), :],
                    w2_ref[...], precision=_P,
                    preferred_element_type=jnp.float32) + b2_ref[...]
        m = jnp.max(u, axis=1, keepdims=True)
        lse = jnp.log(jnp.sum(jnp.exp(u - m), axis=1, keepdims=True)) + m
        out_ref[...] = u - lse
        acc_ref[...] = jnp.zeros_like(acc_ref)


def kernel(x, adj, W1, b1, W2, b2):
    n, nfeat = x.shape
    nhid = W1.shape[1]
    nclass = W2.shape[1]
    b1r = b1.reshape(1, nhid)
    b2r = b2.reshape(1, nclass)

    h_pad, acc = pl.pallas_call(
        _phase_a_body,
        grid=(n // BR,),
        in_specs=[
            pl.BlockSpec((BR, n), lambda i: (i, 0)),
            pl.BlockSpec((n, nfeat), lambda i: (0, 0)),
            pl.BlockSpec((nfeat, nhid), lambda i: (0, 0)),
            pl.BlockSpec((1, nhid), lambda i: (0, 0)),
        ],
        out_specs=[
            pl.BlockSpec((NPAD, nhid), lambda i: (0, 0)),
            pl.BlockSpec((APAD, nhid), lambda i: (0, 0)),
        ],
        out_shape=[
            jax.ShapeDtypeStruct((NPAD, nhid), jnp.float32),
            jax.ShapeDtypeStruct((APAD, nhid), jnp.float32),
        ],
        scratch_shapes=[
            pltpu.VMEM((n, nhid), jnp.float32),
            pltpu.VMEM((BR, nhid), jnp.float32),
        ],
    )(adj, x, W1, b1r)

    sched = jnp.asarray(_build_schedule(n))
    tsteps = sched.shape[1]

    grid_spec = pltpu.PrefetchScalarGridSpec(
        num_scalar_prefetch=1,
        grid=(tsteps,),
        in_specs=[
            pl.BlockSpec((BG, CW), lambda t, s: (s[0, t], s[1, t])),
            pl.BlockSpec((NPAD, nhid), lambda t, s: (0, 0)),
            pl.BlockSpec((APAD, nhid), lambda t, s: (0, 0)),
            pl.BlockSpec((nhid, nclass), lambda t, s: (0, 0)),
            pl.BlockSpec((1, nclass), lambda t, s: (0, 0)),
        ],
        out_specs=pl.BlockSpec((BG, nclass), lambda t, s: (s[0, t], 0)),
        scratch_shapes=[pltpu.VMEM((BG, nhid), jnp.float32)],
    )

    out = pl.pallas_call(
        functools.partial(_phase_b_body, n),
        grid_spec=grid_spec,
        out_shape=jax.ShapeDtypeStruct((n, nclass), jnp.float32),
    )(sched, adj, h_pad, acc, W2, b2r)

    return out
